# baseline (speedup 1.0000x reference)
import jax
import jax.numpy as jnp
from jax import lax
from jax.experimental import pallas as pl
from jax.experimental.pallas import tpu as pltpu
from jax.experimental.pallas import tpu_sc as plsc

B = 1024
V = 1000
VP = 1024
M = 32
H = 256
TIMING_DIM = 10
DECAY = 0.99

_NC, _NS = 2, 16
_NW = _NC * _NS
_ROWS = B * M
_PER_W = _ROWS // _NW
_CHUNK = 32
_NCHUNK = _PER_W // _CHUNK


_BP = 64


def _pad_body(x_ref, mem_ref, mt_ref, ms_ref,
              table_ref, bt_ref, ss_ref, idx_ref):
    i = pl.program_id(0)
    table_ref[:, 0, :V] = x_ref[...]
    table_ref[:, 1:, :V] = mem_ref[:, : M - 1, :]
    table_ref[:, :, V:] = jnp.zeros((_BP, M, VP - V), jnp.float32)

    lane = lax.broadcasted_iota(jnp.int32, (_BP, M), 1)
    a = jnp.where(lane < M - 1, mt_ref[...] + 1, 0)
    news = (lane + 1) & (M - 1)
    c = (a * M + news).astype(jnp.float32)
    k = 2
    while k <= M:
        d = k // 2
        while d >= 1:
            r = lax.broadcasted_iota(jnp.int32, (M, M), 0)
            l2 = lax.broadcasted_iota(jnp.int32, (M, M), 1)
            perm = (r == (l2 ^ d)).astype(jnp.float32)
            cp = jnp.dot(c, perm, preferred_element_type=jnp.float32,
                         precision=lax.Precision.HIGHEST)
            up = (lane & k) == 0
            takemin = up == ((lane & d) == 0)
            c = jnp.where(takemin, jnp.minimum(c, cp), jnp.maximum(c, cp))
            d //= 2
        k *= 2
    ci = c.astype(jnp.int32)
    sidx = ci & (M - 1)
    sti = ci >> 5
    st = sti.astype(jnp.float32)
    bglob = lax.broadcasted_iota(jnp.int32, (_BP, M), 0) + i * _BP
    idx_ref[...] = bglob * M + sidx
    norm = st / (st[:, M - 1:M] + 1.0)
    cols = lax.broadcasted_iota(jnp.int32, (M, 352), 1)
    rows = lax.broadcasted_iota(jnp.int32, (M, 352), 0)
    btv = jnp.dot(norm, (cols == rows + M * TIMING_DIM).astype(jnp.float32),
                  preferred_element_type=jnp.float32,
                  precision=lax.Precision.HIGHEST)
    for t in range(TIMING_DIM):
        et = (cols == rows * TIMING_DIM + t).astype(jnp.float32)
        bit_t = ((sti >> t) & 1).astype(jnp.float32)
        btv += jnp.dot(bit_t, et, preferred_element_type=jnp.float32,
                       precision=lax.Precision.HIGHEST)
    bt_ref[...] = btv
    oh = (sidx[:, :, None] ==
          (lax.broadcasted_iota(jnp.int32, (_BP, M, M), 2) + 1)
          ).astype(jnp.float32)
    ss_ref[...] = DECAY * jnp.sum(oh * ms_ref[...][:, None, :], axis=2)


def _pad_call(x, memory, memory_timings, memory_surprise):
    return pl.pallas_call(
        _pad_body,
        grid=(B // _BP,),
        in_specs=[
            pl.BlockSpec((_BP, V), lambda i: (i, 0)),
            pl.BlockSpec((_BP, M, V), lambda i: (i, 0, 0)),
            pl.BlockSpec((_BP, M), lambda i: (i, 0)),
            pl.BlockSpec((_BP, M), lambda i: (i, 0)),
        ],
        out_specs=[
            pl.BlockSpec((_BP, M, VP), lambda i: (i, 0, 0)),
            pl.BlockSpec((_BP, 352), lambda i: (i, 0)),
            pl.BlockSpec((_BP, M), lambda i: (i, 0)),
            pl.BlockSpec((_BP, M), lambda i: (i, 0)),
        ],
        out_shape=[
            jax.ShapeDtypeStruct((B, M, VP), jnp.float32),
            jax.ShapeDtypeStruct((B, 352), jnp.float32),
            jax.ShapeDtypeStruct((B, M), jnp.float32),
            jax.ShapeDtypeStruct((B, M), jnp.int32),
        ],
        compiler_params=pltpu.CompilerParams(
            dimension_semantics=("arbitrary",)),
    )(x, memory, memory_timings, memory_surprise)


def _sc_gather_body(nrows, idx_hbm, table_hbm, out_hbm, idx_v, rows0, rows1,
                    gsem, wsem):
    per_w = nrows // _NW
    nchunk = per_w // _CHUNK
    wid = lax.axis_index("s") * _NC + lax.axis_index("c")
    base = wid * per_w
    pltpu.sync_copy(idx_hbm.at[pl.ds(base, per_w)], idx_v)
    rows = (rows0, rows1)
    writes = [None] * nchunk
    gathers = [None] * nchunk

    def start_gather(c):
        gathers[c] = pltpu.async_copy(
            table_hbm.at[idx_v.at[pl.ds(c * _CHUNK, _CHUNK)]],
            rows[c % 2], gsem)

    start_gather(0)
    for c in range(nchunk):
        gathers[c].wait()
        if c + 1 < nchunk:
            if c >= 1:
                writes[c - 1].wait()
            start_gather(c + 1)
        writes[c] = pltpu.async_copy(
            rows[c % 2], out_hbm.at[pl.ds(base + c * _CHUNK, _CHUNK)], wsem)
    writes[nchunk - 2].wait()
    writes[nchunk - 1].wait()


def _sc_gather(idx, table):
    nrows = idx.shape[0]
    return pl.kernel(
        lambda *refs: _sc_gather_body(nrows, *refs),
        out_type=jax.ShapeDtypeStruct((nrows, VP), jnp.float32),
        mesh=plsc.VectorSubcoreMesh(core_axis_name="c", subcore_axis_name="s"),
        scratch_types=[
            pltpu.VMEM((nrows // _NW,), jnp.int32),
            pltpu.VMEM((_CHUNK, VP), jnp.float32),
            pltpu.VMEM((_CHUNK, VP), jnp.float32),
            pltpu.SemaphoreType.DMA,
            pltpu.SemaphoreType.DMA,
        ],
    )(idx, table)


_BT = 64
_NB = B // _BT
_NPH = 4
_MH = M // _NPH


def _mlp_body(x_ref, lp_ref, sm_ref, bt_ref, ss_ref,
              w1s_ref, w1bt_ref, w1ss_ref, b1_ref,
              wgs_ref, wgbt_ref, wgss_ref, bg_ref,
              w2_ref, b2_ref, out_ref, acc1_ref, acc2_ref):
    k = pl.program_id(0)
    i = pl.program_id(1)
    a1 = jnp.zeros((_BT, H), jnp.float32)
    a2 = jnp.zeros((_BT, H), jnp.float32)
    for j in range(_MH):
        smj = sm_ref[:, j, :V]
        a1 += jnp.dot(smj, w1s_ref[j], preferred_element_type=jnp.float32)
        a2 += jnp.dot(smj, wgs_ref[j], preferred_element_type=jnp.float32)

    @pl.when(k == 0)
    def _phase0():
        btb = bt_ref[...]
        b1t = a1 + jnp.dot(btb, w1bt_ref[...],
                           preferred_element_type=jnp.float32)
        b2t = a2 + jnp.dot(btb, wgbt_ref[...],
                           preferred_element_type=jnp.float32)
        surprise = -jnp.log(
            jnp.sum(x_ref[...] * lp_ref[...], axis=1, keepdims=True) + 1e-08)
        ss = ss_ref[...]
        b1t += jnp.dot(ss, w1ss_ref[...], preferred_element_type=jnp.float32)
        b2t += jnp.dot(ss, wgss_ref[...], preferred_element_type=jnp.float32)
        b1t += surprise * w1ss_ref[0:1, :]
        b2t += surprise * wgss_ref[0:1, :]
        acc1_ref[i] = b1t
        acc2_ref[i] = b2t

    @pl.when(jnp.logical_and(k > 0, k < _NPH - 1))
    def _mid():
        acc1_ref[i] = acc1_ref[i] + a1
        acc2_ref[i] = acc2_ref[i] + a2

    @pl.when(k == _NPH - 1)
    def _phase1():
        t1 = acc1_ref[i] + a1 + b1_ref[...]
        t2 = acc2_ref[i] + a2 + bg_ref[...]
        h = t1 * jax.nn.sigmoid(t2)
        out_ref[...] = (
            jnp.dot(h, w2_ref[...], preferred_element_type=jnp.float32)
            + b2_ref[...])


def _mlp_call(x, lp, sm, bt, ss, w1s, w1bt, w1ss, b1,
              wgs, wgbt, wgss, bg, w2, b2):
    nb = x.shape[0] // _BT

    def _c(shape):
        return pl.BlockSpec(shape, lambda k, i: (0,) * len(shape))

    return pl.pallas_call(
        _mlp_body,
        grid=(_NPH, nb),
        in_specs=[
            pl.BlockSpec((_BT, V), lambda k, i: (i, 0)),
            pl.BlockSpec((_BT, V), lambda k, i: (i, 0)),
            pl.BlockSpec((_BT, _MH, VP), lambda k, i: (i, k, 0)),
            pl.BlockSpec((_BT, 352), lambda k, i: (i, 0)),
            pl.BlockSpec((_BT, M), lambda k, i: (i, 0)),
            pl.BlockSpec((_MH, V, H), lambda k, i: (k, 0, 0)),
            _c((352, H)), _c((M, H)), _c((1, H)),
            pl.BlockSpec((_MH, V, H), lambda k, i: (k, 0, 0)),
            _c((352, H)), _c((M, H)), _c((1, H)),
            _c((H, V)), _c((1, V)),
        ],
        out_specs=pl.BlockSpec((_BT, V), lambda k, i: (i, 0)),
        out_shape=jax.ShapeDtypeStruct((x.shape[0], V), jnp.float32),
        scratch_shapes=[
            pltpu.VMEM((nb, _BT, H), jnp.float32),
            pltpu.VMEM((nb, _BT, H), jnp.float32),
        ],
        compiler_params=pltpu.CompilerParams(
            dimension_semantics=("arbitrary", "arbitrary")),
    )(x, lp, sm, bt, ss, w1s, w1bt, w1ss, b1,
      wgs, wgbt, wgss, bg, w2, b2)


def kernel(x, memory, memory_timings, memory_surprise, last_prediction,
           W1, b1, Wg, bg, W2, b2):
    table, bt, ss, idxo = _pad_call(x, memory, memory_timings,
                                    memory_surprise)
    table = table.reshape(_ROWS, VP)

    sorted_mem = _sc_gather(idxo.reshape(_ROWS), table).reshape(
        B, M, VP).astype(jnp.bfloat16)

    w1s = W1[:M * V].reshape(M, V, H).astype(jnp.bfloat16)
    wgs = Wg[:M * V].reshape(M, V, H).astype(jnp.bfloat16)
    w1bt = W1[M * V:M * V + 352]
    w1ss = W1[M * V + 352:]
    wgbt = Wg[M * V:M * V + 352]
    wgss = Wg[M * V + 352:]

    return _mlp_call(x, last_prediction, sorted_mem, bt, ss,
                     w1s, w1bt, w1ss, b1.reshape(1, H),
                     wgs, wgbt, wgss, bg.reshape(1, H),
                     W2, b2.reshape(1, V))


# speedup vs baseline: 1.0240x; 1.0240x over previous
import jax
import jax.numpy as jnp
from jax import lax
from jax.experimental import pallas as pl
from jax.experimental.pallas import tpu as pltpu
from jax.experimental.pallas import tpu_sc as plsc

B = 1024
V = 1000
VP = 1024
M = 32
H = 256
TIMING_DIM = 10
DECAY = 0.99

_NC, _NS = 2, 16
_NW = _NC * _NS
_ROWS = B * M
_PER_W = _ROWS // _NW
_CHUNK = 32
_NCHUNK = _PER_W // _CHUNK


_BP = 64


def _pad_body(x_ref, mem_ref, mt_ref, ms_ref,
              table_ref, bt_ref, ss_ref, idx_ref):
    i = pl.program_id(0)
    table_ref[:, 0, :V] = x_ref[...]
    table_ref[:, 1:, :V] = mem_ref[:, : M - 1, :]
    table_ref[:, :, V:] = jnp.zeros((_BP, M, VP - V), jnp.float32)

    lane = lax.broadcasted_iota(jnp.int32, (_BP, M), 1)
    a = jnp.where(lane < M - 1, mt_ref[...] + 1, 0)
    news = (lane + 1) & (M - 1)
    c = (a * M + news).astype(jnp.float32)
    k = 2
    while k <= M:
        d = k // 2
        while d >= 1:
            r = lax.broadcasted_iota(jnp.int32, (M, M), 0)
            l2 = lax.broadcasted_iota(jnp.int32, (M, M), 1)
            perm = (r == (l2 ^ d)).astype(jnp.float32)
            cp = jnp.dot(c, perm, preferred_element_type=jnp.float32,
                         precision=lax.Precision.HIGHEST)
            up = (lane & k) == 0
            takemin = up == ((lane & d) == 0)
            c = jnp.where(takemin, jnp.minimum(c, cp), jnp.maximum(c, cp))
            d //= 2
        k *= 2
    ci = c.astype(jnp.int32)
    sidx = ci & (M - 1)
    sti = ci >> 5
    st = sti.astype(jnp.float32)
    bglob = lax.broadcasted_iota(jnp.int32, (_BP, M), 0) + i * _BP
    idx_ref[...] = bglob * M + sidx
    norm = st / (st[:, M - 1:M] + 1.0)
    cols = lax.broadcasted_iota(jnp.int32, (M, 352), 1)
    rows = lax.broadcasted_iota(jnp.int32, (M, 352), 0)
    btv = jnp.dot(norm, (cols == rows + M * TIMING_DIM).astype(jnp.float32),
                  preferred_element_type=jnp.float32,
                  precision=lax.Precision.HIGHEST)
    for t in range(TIMING_DIM):
        et = (cols == rows * TIMING_DIM + t).astype(jnp.float32)
        bit_t = ((sti >> t) & 1).astype(jnp.float32)
        btv += jnp.dot(bit_t, et, preferred_element_type=jnp.float32,
                       precision=lax.Precision.HIGHEST)
    bt_ref[...] = btv
    oh = (sidx[:, :, None] ==
          (lax.broadcasted_iota(jnp.int32, (_BP, M, M), 2) + 1)
          ).astype(jnp.float32)
    ss_ref[...] = DECAY * jnp.sum(oh * ms_ref[...][:, None, :], axis=2)


def _pad_call(x, memory, memory_timings, memory_surprise):
    return pl.pallas_call(
        _pad_body,
        grid=(B // _BP,),
        in_specs=[
            pl.BlockSpec((_BP, V), lambda i: (i, 0)),
            pl.BlockSpec((_BP, M, V), lambda i: (i, 0, 0)),
            pl.BlockSpec((_BP, M), lambda i: (i, 0)),
            pl.BlockSpec((_BP, M), lambda i: (i, 0)),
        ],
        out_specs=[
            pl.BlockSpec((_BP, M, VP), lambda i: (i, 0, 0)),
            pl.BlockSpec((_BP, 352), lambda i: (i, 0)),
            pl.BlockSpec((_BP, M), lambda i: (i, 0)),
            pl.BlockSpec((_BP, M), lambda i: (i, 0)),
        ],
        out_shape=[
            jax.ShapeDtypeStruct((B, M, VP), jnp.float32),
            jax.ShapeDtypeStruct((B, 352), jnp.float32),
            jax.ShapeDtypeStruct((B, M), jnp.float32),
            jax.ShapeDtypeStruct((B, M), jnp.int32),
        ],
        compiler_params=pltpu.CompilerParams(
            dimension_semantics=("arbitrary",)),
    )(x, memory, memory_timings, memory_surprise)


def _sc_gather_body(nrows, idx_hbm, table_hbm, out_hbm, idx_v, rows0, rows1,
                    gsem, wsem):
    per_w = nrows // _NW
    nchunk = per_w // _CHUNK
    wid = lax.axis_index("s") * _NC + lax.axis_index("c")
    base = wid * per_w
    pltpu.sync_copy(idx_hbm.at[pl.ds(base, per_w)], idx_v)
    rows = (rows0, rows1)
    writes = [None] * nchunk
    gathers = [None] * nchunk

    def start_gather(c):
        gathers[c] = pltpu.async_copy(
            table_hbm.at[idx_v.at[pl.ds(c * _CHUNK, _CHUNK)]],
            rows[c % 2], gsem)

    start_gather(0)
    for c in range(nchunk):
        gathers[c].wait()
        if c + 1 < nchunk:
            if c >= 1:
                writes[c - 1].wait()
            start_gather(c + 1)
        writes[c] = pltpu.async_copy(
            rows[c % 2], out_hbm.at[pl.ds(base + c * _CHUNK, _CHUNK)], wsem)
    writes[nchunk - 2].wait()
    writes[nchunk - 1].wait()


def _sc_gather(idx, table):
    nrows = idx.shape[0]
    return pl.kernel(
        lambda *refs: _sc_gather_body(nrows, *refs),
        out_type=jax.ShapeDtypeStruct((nrows, VP), jnp.float32),
        mesh=plsc.VectorSubcoreMesh(core_axis_name="c", subcore_axis_name="s"),
        scratch_types=[
            pltpu.VMEM((nrows // _NW,), jnp.int32),
            pltpu.VMEM((_CHUNK, VP), jnp.float32),
            pltpu.VMEM((_CHUNK, VP), jnp.float32),
            pltpu.SemaphoreType.DMA,
            pltpu.SemaphoreType.DMA,
        ],
    )(idx, table)


_BT = 64
_NB = B // _BT
_NPH = 4
_MH = M // _NPH


def _mlp_body(x_ref, lp_ref, sm_ref, bt_ref, ss_ref,
              w1s_ref, w1bt_ref, w1ss_ref, b1_ref,
              wgs_ref, wgbt_ref, wgss_ref, bg_ref,
              w2_ref, b2_ref, out_ref, acc1_ref, acc2_ref):
    k = pl.program_id(0)
    i = pl.program_id(1)
    a1 = jnp.zeros((_BT, H), jnp.float32)
    a2 = jnp.zeros((_BT, H), jnp.float32)
    for j in range(_MH):
        smj = sm_ref[:, j, :V]
        a1 += jnp.dot(smj, w1s_ref[j], preferred_element_type=jnp.float32)
        a2 += jnp.dot(smj, wgs_ref[j], preferred_element_type=jnp.float32)

    @pl.when(k == 0)
    def _phase0():
        btb = bt_ref[...]
        b1t = a1 + jnp.dot(btb, w1bt_ref[...],
                           preferred_element_type=jnp.float32)
        b2t = a2 + jnp.dot(btb, wgbt_ref[...],
                           preferred_element_type=jnp.float32)
        surprise = -jnp.log(
            jnp.sum(x_ref[...] * lp_ref[...], axis=1, keepdims=True) + 1e-08)
        ss = ss_ref[...]
        b1t += jnp.dot(ss, w1ss_ref[...], preferred_element_type=jnp.float32)
        b2t += jnp.dot(ss, wgss_ref[...], preferred_element_type=jnp.float32)
        b1t += surprise * w1ss_ref[0:1, :]
        b2t += surprise * wgss_ref[0:1, :]
        acc1_ref[i] = b1t
        acc2_ref[i] = b2t

    @pl.when(jnp.logical_and(k > 0, k < _NPH - 1))
    def _mid():
        acc1_ref[i] = acc1_ref[i] + a1
        acc2_ref[i] = acc2_ref[i] + a2

    @pl.when(k == _NPH - 1)
    def _phase1():
        t1 = acc1_ref[i] + a1 + b1_ref[...]
        t2 = acc2_ref[i] + a2 + bg_ref[...]
        h = t1 * jax.nn.sigmoid(t2)
        out_ref[...] = (
            jnp.dot(h, w2_ref[...], preferred_element_type=jnp.float32)
            + b2_ref[...])


def _mlp_call(x, lp, sm, bt, ss, w1s, w1bt, w1ss, b1,
              wgs, wgbt, wgss, bg, w2, b2):
    nb = x.shape[0] // _BT

    def _c(shape):
        return pl.BlockSpec(shape, lambda k, i: (0,) * len(shape))

    return pl.pallas_call(
        _mlp_body,
        grid=(_NPH, nb),
        in_specs=[
            pl.BlockSpec((_BT, V), lambda k, i: (i, 0)),
            pl.BlockSpec((_BT, V), lambda k, i: (i, 0)),
            pl.BlockSpec((_BT, _MH, VP), lambda k, i: (i, k, 0)),
            pl.BlockSpec((_BT, 352), lambda k, i: (i, 0)),
            pl.BlockSpec((_BT, M), lambda k, i: (i, 0)),
            pl.BlockSpec((_MH, V, H), lambda k, i: (k, 0, 0)),
            _c((352, H)), _c((M, H)), _c((1, H)),
            pl.BlockSpec((_MH, V, H), lambda k, i: (k, 0, 0)),
            _c((352, H)), _c((M, H)), _c((1, H)),
            _c((H, V)), _c((1, V)),
        ],
        out_specs=pl.BlockSpec((_BT, V), lambda k, i: (i, 0)),
        out_shape=jax.ShapeDtypeStruct((x.shape[0], V), jnp.float32),
        scratch_shapes=[
            pltpu.VMEM((nb, _BT, H), jnp.float32),
            pltpu.VMEM((nb, _BT, H), jnp.float32),
        ],
        compiler_params=pltpu.CompilerParams(
            dimension_semantics=("arbitrary", "arbitrary")),
    )(x, lp, sm, bt, ss, w1s, w1bt, w1ss, b1,
      wgs, wgbt, wgss, bg, w2, b2)


def kernel(x, memory, memory_timings, memory_surprise, last_prediction,
           W1, b1, Wg, bg, W2, b2):
    table, bt, ss, idxo = _pad_call(x, memory, memory_timings,
                                    memory_surprise)
    table = table.reshape(_ROWS, VP)

    sorted_mem = _sc_gather(idxo.reshape(_ROWS), table).reshape(B, M, VP)

    w1s = W1[:M * V].reshape(M, V, H)
    wgs = Wg[:M * V].reshape(M, V, H)
    w1bt = W1[M * V:M * V + 352]
    w1ss = W1[M * V + 352:]
    wgbt = Wg[M * V:M * V + 352]
    wgss = Wg[M * V + 352:]

    return _mlp_call(x, last_prediction, sorted_mem, bt, ss,
                     w1s, w1bt, w1ss, b1.reshape(1, H),
                     wgs, wgbt, wgss, bg.reshape(1, H),
                     W2, b2.reshape(1, V))
